# baseline (speedup 1.0000x reference)
import functools
import jax
import jax.numpy as jnp
from jax import lax
from jax.experimental import pallas as pl
from jax.experimental.pallas import tpu as pltpu
from jax.experimental.pallas import tpu_sc as plsc

N = 256
B = 64
K = 200.0

_NC, _NS, _L = 2, 16, 16
_NW = _NC * _NS
_SPW = B // _NW

f32 = jnp.float32


def _sc_body(x_hbm, out_hbm, xv, hist_v, pfx_v, out_v, s0, s1, s2, s3):
    wid = lax.axis_index("s") * _NC + lax.axis_index("c")
    lane = lax.iota(jnp.int32, _L)
    lanef = lane.astype(f32)
    zero16 = jnp.zeros((_L,), f32)
    sems = [s0, s1, s2, s3]

    def _tree(vs):
        while len(vs) > 1:
            nxt = [vs[i] + vs[i + 1] for i in range(0, len(vs) - 1, 2)]
            if len(vs) % 2:
                nxt.append(vs[-1])
            vs = nxt
        return vs[0]

    def start_sample_dma(b):
        return [pltpu.async_copy(
                    x_hbm.at[pl.ds(b * N + 64 * q, 64), :],
                    xv.at[pl.ds(64 * q, 64), :], sems[q])
                for q in range(4)]

    def hist_sample(cps):
        def group(g, carry):
            for q in range(1, 4):
                @pl.when(g == 4 * q)
                def _():
                    cps[q].wait()
            cc = [None] * 16
            cw = [None] * 16
            rc_vec = zero16
            rw_vec = zero16
            for rr in range(16):
                row = g * 16 + rr
                ws = []
                mfs = []
                for k in range(16):
                    v = xv[row, pl.ds(16 * k, 16)]
                    m = v > 0.5
                    mf = jnp.where(m, 1.0, 0.0)
                    w = jnp.where(m, v, 0.0)
                    ws.append(w)
                    mfs.append(mf)
                    cc[k] = mf if rr == 0 else cc[k] + mf
                    cw[k] = w if rr == 0 else cw[k] + w
                sel = (lane == rr).astype(f32)
                rw_vec = rw_vec + sel * jnp.sum(_tree(ws))
                rc_vec = rc_vec + sel * jnp.sum(_tree(mfs))
            for k in range(16):
                sl = pl.ds(16 * k, 16)
                hist_v[sl] = hist_v[sl] + cc[k]
                sl = pl.ds(N + 16 * k, 16)
                hist_v[sl] = hist_v[sl] + cw[k]
            hist_v[pl.ds(2 * N + g * 16, 16)] = rc_vec
            hist_v[pl.ds(3 * N + g * 16, 16)] = rw_vec
            return carry
        cps[0].wait()
        lax.fori_loop(0, 16, group, jnp.int32(0))

    def build_prefix(cnt_off, pfx_off):
        for k in range(16):
            ch = hist_v[pl.ds(cnt_off + 16 * k, 16)]
            pfx_v[pl.ds(pfx_off + 16 * k, 16)] = plsc.cumsum(ch)
        ends = plsc.load_gather(pfx_v, [jnp.int32(pfx_off) + lane * 16 + 15])
        cums = plsc.cumsum(ends)
        total = jnp.max(cums)
        pfx_v[pl.ds(512, 16)] = cums - ends
        for k in range(16):
            bk = plsc.load_gather(pfx_v, [jnp.full((_L,), 512 + k, jnp.int32)])
            sl = pl.ds(pfx_off + 16 * k, 16)
            pfx_v[sl] = pfx_v[sl] + bk
        return total

    def find_cut(cnt_off, pfx_off, total, largest):
        cbest = None
        for k in range(16):
            p = pfx_v[pl.ds(pfx_off + 16 * k, 16)]
            cn = hist_v[pl.ds(cnt_off + 16 * k, 16)]
            jg = lanef + jnp.float32(16 * k)
            if largest:
                cand = jnp.where((total - p + cn) >= K, jg, -1.0)
            else:
                cand = jnp.where(p >= K, jg, 256.0)
            if cbest is None:
                cbest = cand
            elif largest:
                cbest = jnp.maximum(cbest, cand)
            else:
                cbest = jnp.minimum(cbest, cand)
        c = jnp.max(cbest) if largest else -jnp.max(-cbest)
        return jnp.clip(c, 0.0, 255.0)

    def residual_and_full(cnt_off, wsum_off, pfx_off, total, c, largest):
        pcv = zero16
        ccv = zero16
        numv = zero16
        denv = zero16
        for k in range(16):
            jg = lanef + jnp.float32(16 * k)
            oh = (jg == c).astype(f32)
            p = pfx_v[pl.ds(pfx_off + 16 * k, 16)]
            cn = hist_v[pl.ds(cnt_off + 16 * k, 16)]
            wc = hist_v[pl.ds(wsum_off + 16 * k, 16)]
            pcv = pcv + p * oh
            ccv = ccv + cn * oh
            fm = (jg > c).astype(f32) if largest else (jg < c).astype(f32)
            denv = denv + wc * fm
            numv = numv + wc * jg * fm
        pc = jnp.sum(pcv)
        cc = jnp.sum(ccv)
        n_out = (total - pc) if largest else (pc - cc)
        r = K - n_out
        return r, jnp.sum(numv), jnp.sum(denv)

    def prefix_select(load_chunk, r):
        for k in range(16):
            v = load_chunk(k)
            mf = jnp.where(v > 0.5, 1.0, 0.0)
            pfx_v[pl.ds(16 * k, 16)] = plsc.cumsum(mf)
        ends = plsc.load_gather(pfx_v, [lane * 16 + 15])
        cums = plsc.cumsum(ends)
        pfx_v[pl.ds(256, 16)] = cums - ends
        accv = zero16
        for k in range(16):
            v = load_chunk(k)
            rl = pfx_v[pl.ds(16 * k, 16)]
            bk = plsc.load_gather(pfx_v, [jnp.full((_L,), 256 + k, jnp.int32)])
            rank = rl + bk
            take = jnp.logical_and(v > 0.5, rank <= r)
            accv = accv + jnp.where(take, v, 0.0)
        return jnp.sum(accv)

    def col_loader(ci):
        cvec = jnp.zeros((_L,), jnp.int32) + ci
        def load(k):
            return plsc.load_gather(xv, [lane + 16 * k, cvec])
        return load

    def row_loader(ci):
        def load(k):
            return xv[ci, pl.ds(16 * k, 16)]
        return load

    def recip16(d):
        bits = plsc.bitcast(d, jnp.int32)
        y = plsc.bitcast(jnp.int32(0x7EF127EA) - bits, f32)
        for _ in range(4):
            y = y * (2.0 - d * y)
        return y

    def vdiv(num_s, den_s):
        return (zero16 + num_s) * recip16(zero16 + den_s)

    for s in range(_SPW):
        b = wid * _SPW + s
        if s == 0:
            cps = start_sample_dma(b)

        for k in range(64):
            hist_v[pl.ds(16 * k, 16)] = zero16

        hist_sample(cps)

        total = build_prefix(0, 0)
        c_xmax = find_cut(0, 0, total, True)
        c_xmin = find_cut(0, 0, total, False)
        build_prefix(2 * N, N)
        c_ymax = find_cut(2 * N, N, total, True)
        c_ymin = find_cut(2 * N, N, total, False)

        r_xmax, num_xmax, den_xmax = residual_and_full(0, N, 0, total, c_xmax, True)
        r_xmin, num_xmin, den_xmin = residual_and_full(0, N, 0, total, c_xmin, False)
        r_ymax, num_ymax, den_ymax = residual_and_full(2 * N, 3 * N, N, total, c_ymax, True)
        r_ymin, num_ymin, den_ymin = residual_and_full(2 * N, 3 * N, N, total, c_ymin, False)

        pw_xmax = prefix_select(col_loader(c_xmax.astype(jnp.int32)), r_xmax)
        pw_xmin = prefix_select(col_loader(c_xmin.astype(jnp.int32)), r_xmin)
        pw_ymax = prefix_select(row_loader(c_ymax.astype(jnp.int32)), r_ymax)
        pw_ymin = prefix_select(row_loader(c_ymin.astype(jnp.int32)), r_ymin)

        x_max = vdiv(num_xmax + c_xmax * pw_xmax, den_xmax + pw_xmax)
        x_min = vdiv(num_xmin + c_xmin * pw_xmin, den_xmin + pw_xmin)
        y_max = vdiv(num_ymax + c_ymax * pw_ymax, den_ymax + pw_ymax)
        y_min = vdiv(num_ymin + c_ymin * pw_ymin, den_ymin + pw_ymin)

        y_min, y_max = 255.0 - y_max, 255.0 - y_min
        z = 1.0 + y_min * (1.0 / 128.0)
        x_min = x_min - 128.0
        x_max = x_max - 128.0
        inv = recip16(221.0 * z)
        x3min = x_min * inv
        x3max = x_max * inv
        y3min = y_min * inv
        y3max = y_max * inv
        x_size = (x3max - x3min) * 0.5
        y_size = (y3max - y3min) * 0.5
        x_center = (x3max + x3min) * 0.5
        y_center = (y3max + y3min) * 0.5

        vals = jnp.where(lane == 0, x_center,
               jnp.where(lane == 1, y_center,
               jnp.where(lane == 2, z,
               jnp.where(lane == 3, x_size,
               jnp.where(lane == 4, y_size,
               jnp.where(lane == 5, jnp.float32(0.1), jnp.float32(0.0)))))))
        totv = zero16 + total
        out_v[...] = jnp.where(totv > 400.0, vals, jnp.float32(0.0))

        out_cp = pltpu.async_copy(out_v, out_hbm.at[b], sems[0])
        out_cp.wait()
        if s + 1 < _SPW:
            cps = start_sample_dma(b + 1)


@functools.cache
def _sc_stage():
    return pl.kernel(
        _sc_body,
        out_type=jax.ShapeDtypeStruct((B, _L), jnp.float32),
        mesh=plsc.VectorSubcoreMesh(core_axis_name="c", subcore_axis_name="s"),
        compiler_params=pltpu.CompilerParams(needs_layout_passes=False),
        scratch_types=[
            pltpu.VMEM((N, N), jnp.float32),
            pltpu.VMEM((4 * N,), jnp.float32),
            pltpu.VMEM((544,), jnp.float32),
            pltpu.VMEM((_L,), jnp.float32),
            pltpu.SemaphoreType.DMA,
            pltpu.SemaphoreType.DMA,
            pltpu.SemaphoreType.DMA,
            pltpu.SemaphoreType.DMA,
        ],
    )


@jax.jit
def kernel(x):
    out = _sc_stage()(x.reshape(B * N, N))
    return out[:, :7]


# speedup vs baseline: 1.0825x; 1.0825x over previous
import functools
import jax
import jax.numpy as jnp
from jax import lax
from jax.experimental import pallas as pl
from jax.experimental.pallas import tpu as pltpu
from jax.experimental.pallas import tpu_sc as plsc

N = 256
B = 64
K = 200.0

SPB = 8


def _hist_kernel(x_ref, ccnt_ref, cwsum_ref, rcnt_ref, rwsum_ref,
                 rc_scr, rw_scr):
    f32 = jnp.float32
    for s in range(SPB):
        X = x_ref[s]
        m = (X > 0.5).astype(f32)
        w = X * m
        ccnt_ref[pl.ds(s, 1), :] = jnp.sum(m, axis=0, keepdims=True)
        cwsum_ref[pl.ds(s, 1), :] = jnp.sum(w, axis=0, keepdims=True)
        rc_scr[:, pl.ds(s, 1)] = jnp.sum(m, axis=1, keepdims=True)
        rw_scr[:, pl.ds(s, 1)] = jnp.sum(w, axis=1, keepdims=True)
    iu = lax.broadcasted_iota(jnp.int32, (N, N), 0)
    ju = lax.broadcasted_iota(jnp.int32, (N, N), 1)
    ident = (iu == ju).astype(f32)
    hp = jax.lax.Precision.HIGHEST
    rcnt_ref[...] = lax.dot_general(rc_scr[...], ident, (((0,), (0,)), ((), ())))
    rwsum_ref[...] = lax.dot_general(rw_scr[...], ident, (((0,), (0,)), ((), ())),
                                     precision=hp)


def _histograms(x):
    out = pl.pallas_call(
        _hist_kernel,
        grid=(B // SPB,),
        in_specs=[pl.BlockSpec((SPB, N, N), lambda g: (g, 0, 0))],
        out_specs=[pl.BlockSpec((SPB, N), lambda g: (g, 0))] * 4,
        out_shape=[jax.ShapeDtypeStruct((B, N), jnp.float32)] * 4,
        scratch_shapes=[pltpu.VMEM((N, SPB), jnp.float32)] * 2,
        compiler_params=pltpu.CompilerParams(
            dimension_semantics=("arbitrary",),
        ),
    )(x)
    return out


_NC, _NS, _L = 2, 16, 16
_NW = _NC * _NS
_SPW = B // _NW


def _sc_body(xf_hbm, hist_hbm, out_hbm,
             hist_v, pfx_v, lines_v,
             idx0_v, idx1_v, idx2_v, idx3_v, out_v, sem):
    f32 = jnp.float32
    wid = lax.axis_index("s") * _NC + lax.axis_index("c")
    lane = lax.iota(jnp.int32, _L)
    lanef = lane.astype(f32)

    for s in range(_SPW):
        b = wid * _SPW + s

        pltpu.sync_copy(hist_hbm.at[b], hist_v)

        def build_prefix(cnt_off, pfx_off):
            run = jnp.float32(0.0)
            for k in range(16):
                ch = hist_v[pl.ds(cnt_off + 16 * k, 16)]
                pfx_v[pl.ds(pfx_off + 16 * k, 16)] = plsc.cumsum(ch) + run
                run = run + jnp.sum(ch)
            return run

        def find_cut(cnt_off, pfx_off, total, largest):
            c = jnp.float32(-1.0) if largest else jnp.float32(256.0)
            for k in range(16):
                p = pfx_v[pl.ds(pfx_off + 16 * k, 16)]
                cn = hist_v[pl.ds(cnt_off + 16 * k, 16)]
                jg = lanef + jnp.float32(16 * k)
                if largest:
                    cand = jnp.where((total - p + cn) >= K, jg, -1.0)
                    c = jnp.maximum(c, jnp.max(cand))
                else:
                    cand = jnp.where(p >= K, jg, 256.0)
                    c = jnp.minimum(c, jnp.min(cand))
            return jnp.clip(c, 0.0, 255.0)

        def residual_and_full(cnt_off, wsum_off, pfx_off, total, c, largest):
            pc = jnp.float32(0.0)
            cc = jnp.float32(0.0)
            numv = jnp.zeros((_L,), f32)
            denv = jnp.zeros((_L,), f32)
            for k in range(16):
                jg = lanef + jnp.float32(16 * k)
                oh = (jg == c).astype(f32)
                p = pfx_v[pl.ds(pfx_off + 16 * k, 16)]
                cn = hist_v[pl.ds(cnt_off + 16 * k, 16)]
                wc = hist_v[pl.ds(wsum_off + 16 * k, 16)]
                pc = pc + jnp.sum(p * oh)
                cc = cc + jnp.sum(cn * oh)
                fm = (jg > c).astype(f32) if largest else (jg < c).astype(f32)
                denv = denv + wc * fm
                numv = numv + wc * jg * fm
            n_out = (total - pc) if largest else (pc - cc)
            r = K - n_out
            return r, jnp.sum(numv), jnp.sum(denv)

        total = build_prefix(0, 0)
        c_xmax = find_cut(0, 0, total, True)
        c_xmin = find_cut(0, 0, total, False)

        sample_base = b * (N * N)
        copies = []
        for d, (cf, idx_lo, idx_hi) in enumerate(
                [(c_xmax, idx0_v, idx1_v), (c_xmin, idx2_v, idx3_v)]):
            ci = cf.astype(jnp.int32)
            base = sample_base + ci
            for k in range(8):
                idx_lo[pl.ds(16 * k, 16)] = base + N * (lane + 16 * k)
            for k in range(8):
                idx_hi[pl.ds(16 * k, 16)] = base + N * (lane + 16 * k + 128)
            off = d * N
            copies.append(pltpu.async_copy(
                xf_hbm.at[idx_lo], lines_v.at[pl.ds(off, 128)], sem))
            copies.append(pltpu.async_copy(
                xf_hbm.at[idx_hi], lines_v.at[pl.ds(off + 128, 128)], sem))

        build_prefix(512, 256)
        c_ymax = find_cut(512, 256, total, True)
        c_ymin = find_cut(512, 256, total, False)
        for d, cf in enumerate([c_ymax, c_ymin]):
            ci = cf.astype(jnp.int32)
            start = sample_base + ci * N
            copies.append(pltpu.async_copy(
                xf_hbm.at[pl.ds(start, N)], lines_v.at[pl.ds((2 + d) * N, N)],
                sem))

        r_xmax, num_xmax, den_xmax = residual_and_full(0, 256, 0, total, c_xmax, True)
        r_xmin, num_xmin, den_xmin = residual_and_full(0, 256, 0, total, c_xmin, False)
        r_ymax, num_ymax, den_ymax = residual_and_full(512, 768, 256, total, c_ymax, True)
        r_ymin, num_ymin, den_ymin = residual_and_full(512, 768, 256, total, c_ymin, False)

        for cp in copies:
            cp.wait()

        def prefix_select(line_off, r):
            run = jnp.float32(0.0)
            accv = jnp.zeros((_L,), f32)
            for k in range(16):
                v = lines_v[pl.ds(line_off + 16 * k, 16)]
                mf = (v > 0.5).astype(f32)
                rank = plsc.cumsum(mf) + run
                take = jnp.logical_and(mf > 0.0, rank <= r)
                accv = accv + jnp.where(take, v, 0.0)
                run = jnp.max(rank)
            return jnp.sum(accv)

        pw_xmax = prefix_select(0, r_xmax)
        pw_xmin = prefix_select(N, r_xmin)
        pw_ymax = prefix_select(2 * N, r_ymax)
        pw_ymin = prefix_select(3 * N, r_ymin)

        def recip16(d):
            bits = plsc.bitcast(d, jnp.int32)
            y = plsc.bitcast(jnp.int32(0x7EF127EA) - bits, f32)
            for _ in range(4):
                y = y * (2.0 - d * y)
            return y

        def vdiv(num_s, den_s):
            zv = jnp.zeros((_L,), f32)
            return (zv + num_s) * recip16(zv + den_s)

        x_max = vdiv(num_xmax + c_xmax * pw_xmax, den_xmax + pw_xmax)
        x_min = vdiv(num_xmin + c_xmin * pw_xmin, den_xmin + pw_xmin)
        y_max = vdiv(num_ymax + c_ymax * pw_ymax, den_ymax + pw_ymax)
        y_min = vdiv(num_ymin + c_ymin * pw_ymin, den_ymin + pw_ymin)

        y_min, y_max = 255.0 - y_max, 255.0 - y_min
        z = 1.0 + y_min * (1.0 / 128.0)
        x_min = x_min - 128.0
        x_max = x_max - 128.0
        inv = recip16(221.0 * z)
        x3min = x_min * inv
        x3max = x_max * inv
        y3min = y_min * inv
        y3max = y_max * inv
        x_size = (x3max - x3min) * 0.5
        y_size = (y3max - y3min) * 0.5
        x_center = (x3max + x3min) * 0.5
        y_center = (y3max + y3min) * 0.5

        vals = jnp.where(lane == 0, x_center,
               jnp.where(lane == 1, y_center,
               jnp.where(lane == 2, z,
               jnp.where(lane == 3, x_size,
               jnp.where(lane == 4, y_size,
               jnp.where(lane == 5, jnp.float32(0.1), jnp.float32(0.0)))))))
        totv = jnp.zeros((_L,), f32) + total
        out_v[...] = jnp.where(totv > 400.0, vals, jnp.float32(0.0))
        pltpu.sync_copy(out_v, out_hbm.at[b])


@functools.cache
def _sc_stage():
    return pl.kernel(
        _sc_body,
        out_type=jax.ShapeDtypeStruct((B, _L), jnp.float32),
        mesh=plsc.VectorSubcoreMesh(core_axis_name="c", subcore_axis_name="s"),
        compiler_params=pltpu.CompilerParams(needs_layout_passes=False),
        scratch_types=[
            pltpu.VMEM((4 * N,), jnp.float32),
            pltpu.VMEM((2 * N,), jnp.float32),
            pltpu.VMEM((4 * N,), jnp.float32),
            pltpu.VMEM((128,), jnp.int32),
            pltpu.VMEM((128,), jnp.int32),
            pltpu.VMEM((128,), jnp.int32),
            pltpu.VMEM((128,), jnp.int32),
            pltpu.VMEM((_L,), jnp.float32),
            pltpu.SemaphoreType.DMA,
        ],
    )


@jax.jit
def kernel(x):
    ccnt, cwsum, rcnt, rwsum = _histograms(x)
    hist = jnp.concatenate([ccnt, cwsum, rcnt, rwsum], axis=1)
    out = _sc_stage()(x.reshape(-1), hist)
    return out[:, :7]
